# baseline (speedup 1.0000x reference)
import functools

import jax
import jax.numpy as jnp
from jax import lax
from jax.experimental import pallas as pl
from jax.experimental.pallas import tpu as pltpu
from jax.experimental.pallas import tpu_sc as plsc

_L = 16
_NSUB = 16
_NTOK = 200
_NPAD = 256
_DIM = 64
_DBLK = _DIM // _L


def _sc_body(ids_hbm, table_hbm, out_hbm, idx_v, rows_v, acc_v, shared, sem):
    cid = lax.axis_index("c")
    sid = lax.axis_index("s")
    rows_per = _NPAD // _NSUB
    base = sid * rows_per

    pltpu.sync_copy(ids_hbm.at[pl.ds(base, rows_per)], idx_v)
    pltpu.async_copy(table_hbm.at[idx_v], rows_v, sem).wait()

    acc = [jnp.zeros((_L,), jnp.float32) for _ in range(_DBLK)]
    for i in range(rows_per):
        w = jnp.where(base + i < _NTOK, 1.0, 0.0).astype(jnp.float32)
        for d in range(_DBLK):
            acc[d] = acc[d] + rows_v[i, pl.ds(d * _L, _L)] * w
    for d in range(_DBLK):
        acc_v[pl.ds(d * _L, _L)] = acc[d]

    pltpu.sync_copy(acc_v, shared.at[sid])
    plsc.subcore_barrier()

    @pl.when(jnp.logical_and(cid == 0, sid == 0))
    def _():
        pltpu.sync_copy(shared, rows_v)
        tot = [jnp.zeros((_L,), jnp.float32) for _ in range(_DBLK)]
        for s in range(_NSUB):
            for d in range(_DBLK):
                tot[d] = tot[d] + rows_v[s, pl.ds(d * _L, _L)]
        inv_n = jnp.float32(1.0 / _NTOK)
        mean = [t * inv_n for t in tot]
        q = mean[0] * mean[0]
        for d in range(1, _DBLK):
            q = q + mean[d] * mean[d]
        ssq = q[0]
        for i in range(1, _L):
            ssq = ssq + q[i]
        yi = jnp.int32(0x5F3759DF) - (
            lax.bitcast_convert_type(ssq, jnp.int32) >> 1)
        y = lax.bitcast_convert_type(yi, jnp.float32)
        for _ in range(3):
            y = y * (jnp.float32(1.5) - jnp.float32(0.5) * ssq * y * y)
        inv_norm = jnp.minimum(y, jnp.float32(1e12))
        for d in range(_DBLK):
            acc_v[pl.ds(d * _L, _L)] = mean[d] * inv_norm
        pltpu.sync_copy(acc_v, out_hbm)


def _sc_encode(ids_padded, table):
    mesh = plsc.VectorSubcoreMesh(core_axis_name="c", subcore_axis_name="s")
    fn = functools.partial(
        pl.kernel,
        mesh=mesh,
        out_type=jax.ShapeDtypeStruct((_DIM,), jnp.float32),
        scratch_types=[
            pltpu.VMEM((_NPAD // _NSUB,), jnp.int32),
            pltpu.VMEM((_NSUB, _DIM), jnp.float32),
            pltpu.VMEM((_DIM,), jnp.float32),
            pltpu.VMEM_SHARED((_NSUB, _DIM), jnp.float32),
            pltpu.SemaphoreType.DMA,
        ],
        compiler_params=pltpu.CompilerParams(use_tc_tiling_on_sc=False),
    )(_sc_body)
    return fn(ids_padded, table)


def kernel(token_ids, table):
    ids = jnp.zeros((_NPAD,), jnp.int32).at[:_NTOK].set(
        token_ids.astype(jnp.int32))
    return _sc_encode(ids, table)


# speedup vs baseline: 1.7290x; 1.7290x over previous
import functools

import jax
import jax.numpy as jnp
from jax import lax
from jax.experimental import pallas as pl
from jax.experimental.pallas import tpu as pltpu
from jax.experimental.pallas import tpu_sc as plsc

_L = 16
_NSUB = 16
_NTOK = 200
_NPAD = 256
_DIM = 64
_DBLK = _DIM // _L
_ROWS = _NPAD // _NSUB


def _sc_body(ids_hbm, table_hbm, out_hbm, part_hbm,
             idx_v, rows_v, acc_v, red_v, sem):
    cid = lax.axis_index("c")
    sid = lax.axis_index("s")
    base = sid * _ROWS

    @pl.when(cid == 0)
    def _():
        pltpu.sync_copy(ids_hbm.at[pl.ds(base, _ROWS)], idx_v)
        ivec = idx_v[...]
        copies = [
            pltpu.async_copy(
                table_hbm.at[pl.ds(ivec[i], 1)], rows_v.at[pl.ds(i, 1)], sem)
            for i in range(_ROWS)
        ]
        for c in copies:
            c.wait()
        acc = [jnp.zeros((_L,), jnp.float32) for _ in range(_DBLK)]
        for i in range(_ROWS):
            w = jnp.where(base + i < _NTOK, 1.0, 0.0).astype(jnp.float32)
            for d in range(_DBLK):
                acc[d] = acc[d] + rows_v[i, pl.ds(d * _L, _L)] * w
        for d in range(_DBLK):
            acc_v[pl.ds(d * _L, _L)] = acc[d]
        pltpu.sync_copy(acc_v, part_hbm.at[sid])

    plsc.subcore_barrier()

    @pl.when(jnp.logical_and(cid == 0, sid == 0))
    def _():
        pltpu.sync_copy(part_hbm, red_v)
        tot = [jnp.zeros((_L,), jnp.float32) for _ in range(_DBLK)]
        for s in range(_NSUB):
            for d in range(_DBLK):
                tot[d] = tot[d] + red_v[s, pl.ds(d * _L, _L)]
        inv_n = jnp.float32(1.0 / _NTOK)
        mean = [t * inv_n for t in tot]
        q = mean[0] * mean[0]
        for d in range(1, _DBLK):
            q = q + mean[d] * mean[d]
        ssq = q[0]
        for i in range(1, _L):
            ssq = ssq + q[i]
        yi = jnp.int32(0x5F3759DF) - (
            lax.bitcast_convert_type(ssq, jnp.int32) >> 1)
        y = lax.bitcast_convert_type(yi, jnp.float32)
        for _ in range(3):
            y = y * (jnp.float32(1.5) - jnp.float32(0.5) * ssq * y * y)
        inv_norm = jnp.minimum(y, jnp.float32(1e12))
        for d in range(_DBLK):
            acc_v[pl.ds(d * _L, _L)] = mean[d] * inv_norm
        pltpu.sync_copy(acc_v, out_hbm)


def _sc_encode(ids_padded, table):
    mesh = plsc.VectorSubcoreMesh(core_axis_name="c", subcore_axis_name="s")
    fn = functools.partial(
        pl.kernel,
        mesh=mesh,
        out_type=(
            jax.ShapeDtypeStruct((_DIM,), jnp.float32),
            jax.ShapeDtypeStruct((_NSUB, _DIM), jnp.float32),
        ),
        scratch_types=[
            pltpu.VMEM((_ROWS,), jnp.int32),
            pltpu.VMEM((_ROWS, _DIM), jnp.float32),
            pltpu.VMEM((_DIM,), jnp.float32),
            pltpu.VMEM((_NSUB, _DIM), jnp.float32),
            pltpu.SemaphoreType.DMA,
        ],
    )(_sc_body)
    out, _ = fn(ids_padded, table)
    return out


def kernel(token_ids, table):
    ids = jnp.zeros((_NPAD,), jnp.int32).at[:_NTOK].set(
        token_ids.astype(jnp.int32))
    return _sc_encode(ids, table)


# speedup vs baseline: 22.7451x; 13.1547x over previous
import functools

import jax
import jax.numpy as jnp
from jax import lax
from jax.experimental import pallas as pl
from jax.experimental.pallas import tpu as pltpu
from jax.experimental.pallas import tpu_sc as plsc

_L = 16
_NSUB = 16
_NTOK = 200
_NPAD = 256
_DIM = 64
_DBLK = _DIM // _L
_TPW = 13
_LANE = 128


def _sc_body(ids_hbm, tabt_hbm, out_hbm, part_hbm,
             idx_v, bufs, acc_v, red_v, sem):
    cid = lax.axis_index("c")
    sid = lax.axis_index("s")

    @pl.when(cid == 0)
    def _():
        pltpu.sync_copy(ids_hbm, idx_v)
        lane = lax.iota(jnp.int32, _L)
        mine = plsc.load_gather(idx_v, [sid * _TPW + lane])
        rs = [mine[i] for i in range(_TPW)]
        copies = []
        for i in range(_TPW):
            c0 = pl.multiple_of((rs[i] >> 7) << 7, _LANE)
            copies.append(pltpu.async_copy(
                tabt_hbm.at[:, pl.ds(c0, _LANE)], bufs.at[i], sem))
        for c in copies:
            c.wait()
        acc = [jnp.zeros((_L,), jnp.float32) for _ in range(_DBLK)]
        for i in range(_TPW):
            q = jnp.zeros((_L,), jnp.int32) + (rs[i] & (_LANE - 1))
            w = jnp.where(sid * _TPW + i < _NTOK, 1.0, 0.0).astype(
                jnp.float32)
            for d in range(_DBLK):
                v = plsc.load_gather(bufs.at[i], [d * _L + lane, q])
                acc[d] = acc[d] + v * w
        for d in range(_DBLK):
            acc_v[pl.ds(d * _L, _L)] = acc[d]
        pltpu.sync_copy(acc_v, part_hbm.at[sid])

    plsc.subcore_barrier()

    @pl.when(jnp.logical_and(cid == 0, sid == 0))
    def _():
        pltpu.sync_copy(part_hbm, red_v)
        tot = [jnp.zeros((_L,), jnp.float32) for _ in range(_DBLK)]
        for s in range(_NSUB):
            for d in range(_DBLK):
                tot[d] = tot[d] + red_v[s, pl.ds(d * _L, _L)]
        inv_n = jnp.float32(1.0 / _NTOK)
        mean = [t * inv_n for t in tot]
        q = mean[0] * mean[0]
        for d in range(1, _DBLK):
            q = q + mean[d] * mean[d]
        ssq = q[0]
        for i in range(1, _L):
            ssq = ssq + q[i]
        yi = jnp.int32(0x5F3759DF) - (
            lax.bitcast_convert_type(ssq, jnp.int32) >> 1)
        y = lax.bitcast_convert_type(yi, jnp.float32)
        for _ in range(3):
            y = y * (jnp.float32(1.5) - jnp.float32(0.5) * ssq * y * y)
        inv_norm = jnp.minimum(y, jnp.float32(1e12))
        for d in range(_DBLK):
            acc_v[pl.ds(d * _L, _L)] = mean[d] * inv_norm
        pltpu.sync_copy(acc_v, out_hbm)


def _sc_encode(ids_padded, table_t):
    mesh = plsc.VectorSubcoreMesh(core_axis_name="c", subcore_axis_name="s")
    fn = functools.partial(
        pl.kernel,
        mesh=mesh,
        out_type=(
            jax.ShapeDtypeStruct((_DIM,), jnp.float32),
            jax.ShapeDtypeStruct((_NSUB, _DIM), jnp.float32),
        ),
        scratch_types=[
            pltpu.VMEM((_NPAD,), jnp.int32),
            pltpu.VMEM((_TPW, _DIM, _LANE), jnp.float32),
            pltpu.VMEM((_DIM,), jnp.float32),
            pltpu.VMEM((_NSUB, _DIM), jnp.float32),
            pltpu.SemaphoreType.DMA,
        ],
        compiler_params=pltpu.CompilerParams(needs_layout_passes=False),
    )(_sc_body)
    out, _ = fn(ids_padded, table_t)
    return out


def kernel(token_ids, table):
    ids = jnp.zeros((_NPAD,), jnp.int32).at[:_NTOK].set(
        token_ids.astype(jnp.int32))
    return _sc_encode(ids, table.T)


# speedup vs baseline: 22.7766x; 1.0014x over previous
import functools

import jax
import jax.numpy as jnp
from jax import lax
from jax.experimental import pallas as pl
from jax.experimental.pallas import tpu as pltpu
from jax.experimental.pallas import tpu_sc as plsc

_L = 16
_NSUB = 16
_NTOK = 200
_DIM = 64
_DBLK = _DIM // _L
_TPW = 13
_LANE = 128


def _sc_body(ids_hbm, tabt_hbm, out_hbm, part_hbm,
             idx_v, bufs, acc_v, red_v, sem):
    cid = lax.axis_index("c")
    sid = lax.axis_index("s")

    @pl.when(cid == 0)
    def _():
        pltpu.sync_copy(ids_hbm, idx_v)
        lane = lax.iota(jnp.int32, _L)
        slot = jnp.minimum(sid * _TPW + lane, jnp.int32(_NTOK - 1))
        mine = plsc.load_gather(idx_v, [slot])
        rs = [mine[i] for i in range(_TPW)]
        copies = []
        for i in range(_TPW):
            c0 = pl.multiple_of((rs[i] >> 7) << 7, _LANE)
            copies.append(pltpu.async_copy(
                tabt_hbm.at[:, pl.ds(c0, _LANE)], bufs.at[i], sem))
        for c in copies:
            c.wait()
        acc = [jnp.zeros((_L,), jnp.float32) for _ in range(_DBLK)]
        for i in range(_TPW):
            q = jnp.zeros((_L,), jnp.int32) + (rs[i] & (_LANE - 1))
            w = jnp.where(sid * _TPW + i < _NTOK, 1.0, 0.0).astype(
                jnp.float32)
            for d in range(_DBLK):
                v = plsc.load_gather(bufs.at[i], [d * _L + lane, q])
                acc[d] = acc[d] + v * w
        for d in range(_DBLK):
            acc_v[pl.ds(d * _L, _L)] = acc[d]
        pltpu.sync_copy(acc_v, part_hbm.at[sid])

    plsc.subcore_barrier()

    @pl.when(jnp.logical_and(cid == 0, sid == 0))
    def _():
        pltpu.sync_copy(part_hbm, red_v)
        tot = [jnp.zeros((_L,), jnp.float32) for _ in range(_DBLK)]
        for s in range(_NSUB):
            for d in range(_DBLK):
                tot[d] = tot[d] + red_v[s, pl.ds(d * _L, _L)]
        inv_n = jnp.float32(1.0 / _NTOK)
        mean = [t * inv_n for t in tot]
        q = mean[0] * mean[0]
        for d in range(1, _DBLK):
            q = q + mean[d] * mean[d]
        ssq = q[0]
        for i in range(1, _L):
            ssq = ssq + q[i]
        yi = jnp.int32(0x5F3759DF) - (
            lax.bitcast_convert_type(ssq, jnp.int32) >> 1)
        y = lax.bitcast_convert_type(yi, jnp.float32)
        for _ in range(3):
            y = y * (jnp.float32(1.5) - jnp.float32(0.5) * ssq * y * y)
        inv_norm = jnp.minimum(y, jnp.float32(1e12))
        for d in range(_DBLK):
            acc_v[pl.ds(d * _L, _L)] = mean[d] * inv_norm
        pltpu.sync_copy(acc_v, out_hbm)


def _sc_encode(ids_padded, table_t):
    mesh = plsc.VectorSubcoreMesh(core_axis_name="c", subcore_axis_name="s")
    fn = functools.partial(
        pl.kernel,
        mesh=mesh,
        out_type=(
            jax.ShapeDtypeStruct((_DIM,), jnp.float32),
            jax.ShapeDtypeStruct((_NSUB, _DIM), jnp.float32),
        ),
        scratch_types=[
            pltpu.VMEM((_NTOK,), jnp.int32),
            pltpu.VMEM((_TPW, _DIM, _LANE), jnp.float32),
            pltpu.VMEM((_DIM,), jnp.float32),
            pltpu.VMEM((_NSUB, _DIM), jnp.float32),
            pltpu.SemaphoreType.DMA,
        ],
        compiler_params=pltpu.CompilerParams(needs_layout_passes=False),
    )(_sc_body)
    out, _ = fn(ids_padded, table_t)
    return out


def kernel(token_ids, table):
    return _sc_encode(token_ids.astype(jnp.int32), table.T)


# speedup vs baseline: 23.5516x; 1.0340x over previous
import functools

import jax
import jax.numpy as jnp
from jax import lax
from jax.experimental import pallas as pl
from jax.experimental.pallas import tpu as pltpu
from jax.experimental.pallas import tpu_sc as plsc

_L = 16
_NW = 32
_NTOK = 200
_DIM = 64
_DBLK = _DIM // _L
_TPW = 7
_LANE = 128


def _sc_body(ids_hbm, tabt_hbm, part_hbm, idx_v, bufs, acc_v, sem):
    cid = lax.axis_index("c")
    sid = lax.axis_index("s")
    wid = sid * 2 + cid

    pltpu.sync_copy(ids_hbm, idx_v)
    lane = lax.iota(jnp.int32, _L)
    slot = jnp.minimum(wid * _TPW + lane, jnp.int32(_NTOK - 1))
    mine = plsc.load_gather(idx_v, [slot])
    rs = [mine[i] for i in range(_TPW)]
    copies = []
    for i in range(_TPW):
        c0 = pl.multiple_of((rs[i] >> 7) << 7, _LANE)
        copies.append(pltpu.async_copy(
            tabt_hbm.at[:, pl.ds(c0, _LANE)], bufs.at[i], sem))
    for c in copies:
        c.wait()
    acc = [jnp.zeros((_L,), jnp.float32) for _ in range(_DBLK)]
    for i in range(_TPW):
        q = jnp.zeros((_L,), jnp.int32) + (rs[i] & (_LANE - 1))
        w = jnp.where(wid * _TPW + i < _NTOK, 1.0, 0.0).astype(jnp.float32)
        for d in range(_DBLK):
            v = plsc.load_gather(bufs.at[i], [d * _L + lane, q])
            acc[d] = acc[d] + v * w
    for d in range(_DBLK):
        acc_v[pl.ds(d * _L, _L)] = acc[d]
    pltpu.sync_copy(acc_v, part_hbm.at[wid])


def _tc_finish(part_ref, out_ref):
    s = jnp.sum(part_ref[...], axis=0, keepdims=True)
    t = s * jnp.float32(1.0 / _NTOK)
    norm = jnp.maximum(jnp.sqrt(jnp.sum(t * t)), jnp.float32(1e-12))
    out_ref[...] = t / norm


def _sc_encode(ids, table_t):
    mesh = plsc.VectorSubcoreMesh(core_axis_name="c", subcore_axis_name="s")
    gather = functools.partial(
        pl.kernel,
        mesh=mesh,
        out_type=jax.ShapeDtypeStruct((_NW, _DIM), jnp.float32),
        scratch_types=[
            pltpu.VMEM((_NTOK,), jnp.int32),
            pltpu.VMEM((_TPW, _DIM, _LANE), jnp.float32),
            pltpu.VMEM((_DIM,), jnp.float32),
            pltpu.SemaphoreType.DMA,
        ],
        compiler_params=pltpu.CompilerParams(needs_layout_passes=False),
    )(_sc_body)
    parts = gather(ids, table_t)
    out = pl.pallas_call(
        _tc_finish,
        out_shape=jax.ShapeDtypeStruct((1, _DIM), jnp.float32),
    )(parts)
    return out.reshape((_DIM,))


def kernel(token_ids, table):
    return _sc_encode(token_ids.astype(jnp.int32), table.T)
